# Initial kernel scaffold; baseline (speedup 1.0000x reference)
import jax
import jax.numpy as jnp
from jax.experimental import pallas as pl


def kernel(user, item, user_factors, item_factors, W1, b1, W2, b2, W3, b3):
    raise NotImplementedError("write your pallas kernel here")


# speedup vs baseline: 2.8157x; 2.8157x over previous
import functools

import jax
import jax.numpy as jnp
from jax import lax
from jax.experimental import pallas as pl
from jax.experimental.pallas import tpu as pltpu
from jax.experimental.pallas import tpu_sc as plsc

BATCH = 16384
D = 128
GATHER_WINDOW = 128
MLP_BLOCK = 2048


def _gather_body(uf_hbm, if_hbm, ui_hbm, ii_hbm, ue_hbm, ie_hbm):
    def body(ui_vmem, ii_vmem, ue_vmem, ie_vmem):
        pltpu.sync_copy(uf_hbm.at[ui_vmem.at[0]], ue_vmem)
        pltpu.sync_copy(if_hbm.at[ii_vmem.at[0]], ie_vmem)

    pltpu.emit_pipeline(
        body,
        grid=(BATCH // GATHER_WINDOW,),
        in_specs=[
            pl.BlockSpec((1, GATHER_WINDOW), lambda i: (0, i)),
            pl.BlockSpec((1, GATHER_WINDOW), lambda i: (0, i)),
        ],
        out_specs=[
            pl.BlockSpec((GATHER_WINDOW, D), lambda i: (i, 0)),
            pl.BlockSpec((GATHER_WINDOW, D), lambda i: (i, 0)),
        ],
        core_axis_name=("c", "s"),
        dimension_semantics=(pltpu.PARALLEL,),
    )(ui_hbm, ii_hbm, ue_hbm, ie_hbm)


def _sc_gather(user_factors, item_factors, user2d, item2d):
    mesh = plsc.VectorSubcoreMesh(core_axis_name="c", subcore_axis_name="s")
    f = pl.kernel(
        _gather_body,
        out_type=(
            jax.ShapeDtypeStruct((BATCH, D), jnp.float32),
            jax.ShapeDtypeStruct((BATCH, D), jnp.float32),
        ),
        mesh=mesh,
    )
    return f(user_factors, item_factors, user2d, item2d)


def _mlp_body(ue_ref, ie_ref, w1u_ref, w1i_ref, b1_ref, w2_ref, b2_ref,
              w3_ref, b3_ref, o_ref):
    dn = (((1,), (1,)), ((), ()))
    h = lax.dot_general(ue_ref[...], w1u_ref[...], dn,
                        preferred_element_type=jnp.float32)
    h = h + lax.dot_general(ie_ref[...], w1i_ref[...], dn,
                            preferred_element_type=jnp.float32)
    h = jnp.maximum(h + b1_ref[...][None, :], 0.0)
    h = lax.dot_general(h, w2_ref[...], dn, preferred_element_type=jnp.float32)
    h = jnp.maximum(h + b2_ref[...][None, :], 0.0)
    h = lax.dot_general(h, w3_ref[...], dn, preferred_element_type=jnp.float32)
    o_ref[...] = jax.nn.sigmoid(h + b3_ref[...][None, :])


def _tc_mlp(ue, ie, W1u, W1i, b1, W2, b2, W3, b3):
    nblk = BATCH // MLP_BLOCK
    full = lambda shape: pl.BlockSpec(shape, lambda i: tuple(0 for _ in shape))
    return pl.pallas_call(
        _mlp_body,
        grid=(nblk,),
        in_specs=[
            pl.BlockSpec((MLP_BLOCK, D), lambda i: (i, 0)),
            pl.BlockSpec((MLP_BLOCK, D), lambda i: (i, 0)),
            full(W1u.shape), full(W1i.shape), full(b1.shape),
            full(W2.shape), full(b2.shape), full(W3.shape), full(b3.shape),
        ],
        out_specs=pl.BlockSpec((MLP_BLOCK, 2), lambda i: (i, 0)),
        out_shape=jax.ShapeDtypeStruct((BATCH, 2), jnp.float32),
    )(ue, ie, W1u, W1i, b1, W2, b2, W3, b3)


def kernel(user, item, user_factors, item_factors, W1, b1, W2, b2, W3, b3):
    user2d = user.astype(jnp.int32).reshape(1, BATCH)
    item2d = item.astype(jnp.int32).reshape(1, BATCH)
    ue, ie = _sc_gather(user_factors, item_factors, user2d, item2d)
    W1u = W1[:, :D]
    W1i = W1[:, D:]
    return _tc_mlp(ue, ie, W1u, W1i, b1, W2, b2, W3, b3)


# speedup vs baseline: 2.9570x; 1.0502x over previous
import functools

import jax
import jax.numpy as jnp
from jax import lax
from jax.experimental import pallas as pl
from jax.experimental.pallas import tpu as pltpu
from jax.experimental.pallas import tpu_sc as plsc

BATCH = 16384
D = 128
GATHER_WINDOW = 128
MLP_BLOCK = 2048


def _gather_body(uf_hbm, if_hbm, ui_hbm, ii_hbm, ue_hbm, ie_hbm):
    def body(ui_vmem, ii_vmem, ue_vmem, ie_vmem):
        def scoped(s1, s2):
            c1 = pltpu.make_async_copy(uf_hbm.at[ui_vmem.at[0]], ue_vmem, s1)
            c2 = pltpu.make_async_copy(if_hbm.at[ii_vmem.at[0]], ie_vmem, s2)
            c1.start()
            c2.start()
            c1.wait()
            c2.wait()

        pl.run_scoped(scoped, pltpu.SemaphoreType.DMA, pltpu.SemaphoreType.DMA)

    pltpu.emit_pipeline(
        body,
        grid=(BATCH // GATHER_WINDOW,),
        in_specs=[
            pl.BlockSpec((1, GATHER_WINDOW), lambda i: (0, i)),
            pl.BlockSpec((1, GATHER_WINDOW), lambda i: (0, i)),
        ],
        out_specs=[
            pl.BlockSpec((GATHER_WINDOW, D), lambda i: (i, 0)),
            pl.BlockSpec((GATHER_WINDOW, D), lambda i: (i, 0)),
        ],
        core_axis_name=("c", "s"),
        dimension_semantics=(pltpu.PARALLEL,),
    )(ui_hbm, ii_hbm, ue_hbm, ie_hbm)


def _sc_gather(user_factors, item_factors, user2d, item2d):
    mesh = plsc.VectorSubcoreMesh(core_axis_name="c", subcore_axis_name="s")
    f = pl.kernel(
        _gather_body,
        out_type=(
            jax.ShapeDtypeStruct((BATCH, D), jnp.float32),
            jax.ShapeDtypeStruct((BATCH, D), jnp.float32),
        ),
        mesh=mesh,
    )
    return f(user_factors, item_factors, user2d, item2d)


def _mlp_body(ue_ref, ie_ref, w1u_ref, w1i_ref, b1_ref, w2_ref, b2_ref,
              w3_ref, b3_ref, o_ref):
    dn = (((1,), (1,)), ((), ()))
    h = lax.dot_general(ue_ref[...], w1u_ref[...], dn,
                        preferred_element_type=jnp.float32)
    h = h + lax.dot_general(ie_ref[...], w1i_ref[...], dn,
                            preferred_element_type=jnp.float32)
    h = jnp.maximum(h + b1_ref[...][None, :], 0.0)
    h = lax.dot_general(h, w2_ref[...], dn, preferred_element_type=jnp.float32)
    h = jnp.maximum(h + b2_ref[...][None, :], 0.0)
    h = lax.dot_general(h, w3_ref[...], dn, preferred_element_type=jnp.float32)
    o_ref[...] = jax.nn.sigmoid(h + b3_ref[...][None, :])


def _tc_mlp(ue, ie, W1u, W1i, b1, W2, b2, W3, b3):
    nblk = BATCH // MLP_BLOCK
    full = lambda shape: pl.BlockSpec(shape, lambda i: tuple(0 for _ in shape))
    return pl.pallas_call(
        _mlp_body,
        grid=(nblk,),
        in_specs=[
            pl.BlockSpec((MLP_BLOCK, D), lambda i: (i, 0)),
            pl.BlockSpec((MLP_BLOCK, D), lambda i: (i, 0)),
            full(W1u.shape), full(W1i.shape), full(b1.shape),
            full(W2.shape), full(b2.shape), full(W3.shape), full(b3.shape),
        ],
        out_specs=pl.BlockSpec((MLP_BLOCK, 2), lambda i: (i, 0)),
        out_shape=jax.ShapeDtypeStruct((BATCH, 2), jnp.float32),
    )(ue, ie, W1u, W1i, b1, W2, b2, W3, b3)


def kernel(user, item, user_factors, item_factors, W1, b1, W2, b2, W3, b3):
    user2d = user.astype(jnp.int32).reshape(1, BATCH)
    item2d = item.astype(jnp.int32).reshape(1, BATCH)
    ue, ie = _sc_gather(user_factors, item_factors, user2d, item2d)
    W1u = W1[:, :D]
    W1i = W1[:, D:]
    return _tc_mlp(ue, ie, W1u, W1i, b1, W2, b2, W3, b3)
